# baseline (speedup 1.0000x reference)
import functools

import jax
import jax.numpy as jnp
from jax import lax
from jax.experimental import pallas as pl
from jax.experimental.pallas import tpu as pltpu
from jax.experimental.pallas import tpu_sc as plsc

N_ATOMS = 204800
N_ROWS = 86
Z_EMB = 128
PERIOD_EMB = 64
GROUP_EMB = 64
N_PERIODS = 8
N_GROUPS = 20
D_OUT = Z_EMB + PERIOD_EMB + GROUP_EMB

_NC, _NS = 2, 16
_NW = _NC * _NS
_CHUNK = 128
_K = 8
_PRE = 32


def _fuse_body(pm_ref, gm_ref, zt_ref, pt_ref, gt_ref, out_ref):
    pm = pm_ref[...]
    gm = gm_ref[...]
    per_oh = (pm == lax.broadcasted_iota(jnp.int32, (N_ROWS, N_PERIODS), 1)
              ).astype(jnp.float32)
    grp_oh = (gm == lax.broadcasted_iota(jnp.int32, (N_ROWS, N_GROUPS), 1)
              ).astype(jnp.float32)
    h_per = jnp.dot(per_oh, pt_ref[...], preferred_element_type=jnp.float32)
    h_grp = jnp.dot(grp_oh, gt_ref[...], preferred_element_type=jnp.float32)
    out_ref[...] = jnp.concatenate([zt_ref[...], h_per, h_grp], axis=-1)


def _fuse_tables(period_mapping, group_mapping, z_table, period_table,
                 group_table):
    return pl.pallas_call(
        _fuse_body,
        out_shape=jax.ShapeDtypeStruct((N_ROWS, D_OUT), jnp.float32),
    )(period_mapping.reshape(N_ROWS, 1), group_mapping.reshape(N_ROWS, 1),
      z_table, period_table, group_table)


@functools.lru_cache(maxsize=None)
def _make_gather(n_atoms):
    assert n_atoms % (_NW * _CHUNK) == 0
    rows_per_w = n_atoms // _NW
    n_chunks = rows_per_w // _CHUNK
    assert (n_chunks - 5) % 3 == 0
    n_super = (n_chunks - 5) // 3
    mesh = plsc.VectorSubcoreMesh(core_axis_name="c", subcore_axis_name="s")

    @functools.partial(
        pl.kernel,
        out_type=jax.ShapeDtypeStruct((n_atoms, D_OUT), jnp.float32),
        name="sc_embed_gather",
        mesh=mesh,
        compiler_params=pltpu.CompilerParams(needs_layout_passes=False),
        scratch_types=[
            pltpu.VMEM((rows_per_w + 16,), jnp.int32),
            pltpu.VMEM((N_ROWS, D_OUT), jnp.float32),
            pltpu.VMEM((_CHUNK, D_OUT), jnp.float32),
            pltpu.VMEM((_CHUNK, D_OUT), jnp.float32),
            pltpu.VMEM((_CHUNK, D_OUT), jnp.float32),
            pltpu.SemaphoreType.DMA,
            pltpu.SemaphoreType.DMA,
            pltpu.SemaphoreType.DMA,
            pltpu.SemaphoreType.DMA,
            pltpu.SemaphoreType.DMA,
            pltpu.SemaphoreType.DMA,
        ],
    )
    def gather(z_hbm, fused_rep_hbm, fused_hbm, out_hbm, idx_v, table_v,
               rows0, rows1, rows2, sg0, sg1, sg2, sw0, sw1, sw2):
        wid = lax.axis_index("s") * _NC + lax.axis_index("c")
        base = wid * rows_per_w
        rows = (rows0, rows1, rows2)
        sg = (sg0, sg1, sg2)
        sw = (sw0, sw1, sw2)

        def gather_desc(g, b):
            return pltpu.make_async_copy(
                fused_rep_hbm.at[idx_v.at[pl.ds(g * _CHUNK, _PRE)]],
                rows[b].at[pl.ds(0, _PRE)], sg[b])

        def write_desc(g, b):
            return pltpu.make_async_copy(
                rows[b], out_hbm.at[pl.ds(base + g * _CHUNK, _CHUNK)],
                sw[b])

        def fill(g, b):
            @plsc.parallel_loop(_PRE, _CHUNK, unroll=4)
            def fbody(a):
                zl = idx_v[pl.ds(g * _CHUNK + a, 16)]
                zi = (zl[0] >> 3) - wid * N_ROWS
                for kk in range(D_OUT // 16):
                    sl = pl.ds(kk * 16, 16)
                    rows[b][a, sl] = table_v[zi, sl]

        pltpu.sync_copy(fused_hbm, table_v)
        pltpu.sync_copy(z_hbm.at[pl.ds(base, rows_per_w)], idx_v.at[
            pl.ds(0, rows_per_w)])
        off = wid * (N_ROWS * _K)
        pat = lax.iota(jnp.int32, 16) & (_K - 1)

        def addoff(i, carry):
            sl = pl.ds(i * 16, 16)
            idx_v[sl] = idx_v[sl] * _K + (pat + off)
            return carry

        lax.fori_loop(0, rows_per_w // 16, addoff, 0)

        gather_desc(0, 0).start()
        fill(0, 0)
        gather_desc(1, 1).start()
        fill(1, 1)

        gather_desc(0, 0).wait()
        write_desc(0, 0).start()
        gather_desc(2, 2).start()
        fill(2, 2)

        def body(s, carry):
            for j in range(3):
                g = 1 + 3 * s + j
                b = (1 + j) % 3
                gather_desc(g, b).wait()
                write_desc(g, b).start()
                write_desc(g - 1, j).wait()
                gather_desc(g + 2, j).start()
                fill(g + 2, j)
            return carry

        lax.fori_loop(0, n_super, body, 0)

        for t in range(2):
            g = n_chunks - 4 + t
            b = g % 3
            gather_desc(g, b).wait()
            write_desc(g, b).start()
            write_desc(g - 1, (g - 1) % 3).wait()
            gather_desc(g + 2, (g - 1) % 3).start()
            fill(g + 2, (g - 1) % 3)

        for t in range(2):
            g = n_chunks - 2 + t
            b = g % 3
            gather_desc(g, b).wait()
            write_desc(g, b).start()

        for g in (n_chunks - 3, n_chunks - 2, n_chunks - 1):
            write_desc(g, g % 3).wait()

    return gather


def kernel(z, period_mapping, group_mapping, z_table, period_table,
           group_table):
    fused = _fuse_tables(period_mapping, group_mapping, z_table,
                         period_table, group_table)
    fused_rep = jnp.tile(jnp.repeat(fused, _K, axis=0), (_NW, 1))
    return _make_gather(N_ATOMS)(z, fused_rep, fused)


# speedup vs baseline: 1.0761x; 1.0761x over previous
import functools

import jax
import jax.numpy as jnp
from jax import lax
from jax.experimental import pallas as pl
from jax.experimental.pallas import tpu as pltpu
from jax.experimental.pallas import tpu_sc as plsc

N_ATOMS = 204800
N_ROWS = 86
Z_EMB = 128
PERIOD_EMB = 64
GROUP_EMB = 64
N_PERIODS = 8
N_GROUPS = 20
D_OUT = Z_EMB + PERIOD_EMB + GROUP_EMB

_NC, _NS = 2, 16
_NW = _NC * _NS
_CHUNK = 128
_K = 8
_PRE = 16


def _fuse_body(pm_ref, gm_ref, zt_ref, pt_ref, gt_ref, out_ref):
    pm = pm_ref[...]
    gm = gm_ref[...]
    per_oh = (pm == lax.broadcasted_iota(jnp.int32, (N_ROWS, N_PERIODS), 1)
              ).astype(jnp.float32)
    grp_oh = (gm == lax.broadcasted_iota(jnp.int32, (N_ROWS, N_GROUPS), 1)
              ).astype(jnp.float32)
    h_per = jnp.dot(per_oh, pt_ref[...], preferred_element_type=jnp.float32)
    h_grp = jnp.dot(grp_oh, gt_ref[...], preferred_element_type=jnp.float32)
    out_ref[...] = jnp.concatenate([zt_ref[...], h_per, h_grp], axis=-1)


def _fuse_tables(period_mapping, group_mapping, z_table, period_table,
                 group_table):
    return pl.pallas_call(
        _fuse_body,
        out_shape=jax.ShapeDtypeStruct((N_ROWS, D_OUT), jnp.float32),
    )(period_mapping.reshape(N_ROWS, 1), group_mapping.reshape(N_ROWS, 1),
      z_table, period_table, group_table)


@functools.lru_cache(maxsize=None)
def _make_gather(n_atoms):
    assert n_atoms % (_NW * _CHUNK) == 0
    rows_per_w = n_atoms // _NW
    n_chunks = rows_per_w // _CHUNK
    assert (n_chunks - 5) % 3 == 0
    n_super = (n_chunks - 5) // 3
    mesh = plsc.VectorSubcoreMesh(core_axis_name="c", subcore_axis_name="s")

    @functools.partial(
        pl.kernel,
        out_type=jax.ShapeDtypeStruct((n_atoms, D_OUT), jnp.float32),
        name="sc_embed_gather",
        mesh=mesh,
        compiler_params=pltpu.CompilerParams(needs_layout_passes=False),
        scratch_types=[
            pltpu.VMEM((rows_per_w + 16,), jnp.int32),
            pltpu.VMEM((N_ROWS, D_OUT), jnp.float32),
            pltpu.VMEM((_CHUNK, D_OUT), jnp.float32),
            pltpu.VMEM((_CHUNK, D_OUT), jnp.float32),
            pltpu.VMEM((_CHUNK, D_OUT), jnp.float32),
            pltpu.SemaphoreType.DMA,
            pltpu.SemaphoreType.DMA,
            pltpu.SemaphoreType.DMA,
            pltpu.SemaphoreType.DMA,
            pltpu.SemaphoreType.DMA,
            pltpu.SemaphoreType.DMA,
        ],
    )
    def gather(z_hbm, fused_rep_hbm, fused_hbm, out_hbm, idx_v, table_v,
               rows0, rows1, rows2, sg0, sg1, sg2, sw0, sw1, sw2):
        wid = lax.axis_index("s") * _NC + lax.axis_index("c")
        base = wid * rows_per_w
        rows = (rows0, rows1, rows2)
        sg = (sg0, sg1, sg2)
        sw = (sw0, sw1, sw2)

        def gather_desc(g, b):
            return pltpu.make_async_copy(
                fused_rep_hbm.at[idx_v.at[pl.ds(g * _CHUNK, _PRE)]],
                rows[b].at[pl.ds(0, _PRE)], sg[b])

        def write_desc(g, b):
            return pltpu.make_async_copy(
                rows[b], out_hbm.at[pl.ds(base + g * _CHUNK, _CHUNK)],
                sw[b])

        def fill(g, b):
            @plsc.parallel_loop(_PRE, _CHUNK, unroll=4)
            def fbody(a):
                zl = idx_v[pl.ds(g * _CHUNK + a, 16)]
                zi = (zl[0] >> 3) - wid * N_ROWS
                for kk in range(D_OUT // 16):
                    sl = pl.ds(kk * 16, 16)
                    rows[b][a, sl] = table_v[zi, sl]

        pltpu.sync_copy(fused_hbm, table_v)
        pltpu.sync_copy(z_hbm.at[pl.ds(base, rows_per_w)], idx_v.at[
            pl.ds(0, rows_per_w)])
        off = wid * (N_ROWS * _K)
        pat = lax.iota(jnp.int32, 16) & (_K - 1)

        def addoff(i, carry):
            sl = pl.ds(i * 16, 16)
            idx_v[sl] = idx_v[sl] * _K + (pat + off)
            return carry

        lax.fori_loop(0, rows_per_w // 16, addoff, 0)

        gather_desc(0, 0).start()
        fill(0, 0)
        gather_desc(1, 1).start()
        fill(1, 1)

        gather_desc(0, 0).wait()
        write_desc(0, 0).start()
        gather_desc(2, 2).start()
        fill(2, 2)

        def body(s, carry):
            for j in range(3):
                g = 1 + 3 * s + j
                b = (1 + j) % 3
                gather_desc(g, b).wait()
                write_desc(g, b).start()
                write_desc(g - 1, j).wait()
                gather_desc(g + 2, j).start()
                fill(g + 2, j)
            return carry

        lax.fori_loop(0, n_super, body, 0)

        for t in range(2):
            g = n_chunks - 4 + t
            b = g % 3
            gather_desc(g, b).wait()
            write_desc(g, b).start()
            write_desc(g - 1, (g - 1) % 3).wait()
            gather_desc(g + 2, (g - 1) % 3).start()
            fill(g + 2, (g - 1) % 3)

        for t in range(2):
            g = n_chunks - 2 + t
            b = g % 3
            gather_desc(g, b).wait()
            write_desc(g, b).start()

        for g in (n_chunks - 3, n_chunks - 2, n_chunks - 1):
            write_desc(g, g % 3).wait()

    return gather


def kernel(z, period_mapping, group_mapping, z_table, period_table,
           group_table):
    fused = _fuse_tables(period_mapping, group_mapping, z_table,
                         period_table, group_table)
    fused_rep = jnp.tile(jnp.repeat(fused, _K, axis=0), (_NW, 1))
    return _make_gather(N_ATOMS)(z, fused_rep, fused)


# speedup vs baseline: 1.1296x; 1.0498x over previous
import functools

import jax
import jax.numpy as jnp
from jax import lax
from jax.experimental import pallas as pl
from jax.experimental.pallas import tpu as pltpu
from jax.experimental.pallas import tpu_sc as plsc

N_ATOMS = 204800
N_ROWS = 86
Z_EMB = 128
PERIOD_EMB = 64
GROUP_EMB = 64
N_PERIODS = 8
N_GROUPS = 20
D_OUT = Z_EMB + PERIOD_EMB + GROUP_EMB

_NC, _NS = 2, 16
_NW = _NC * _NS
_CHUNK = 128
_K = 8
_PRE = 8


def _fuse_body(pm_ref, gm_ref, zt_ref, pt_ref, gt_ref, out_ref):
    pm = pm_ref[...]
    gm = gm_ref[...]
    per_oh = (pm == lax.broadcasted_iota(jnp.int32, (N_ROWS, N_PERIODS), 1)
              ).astype(jnp.float32)
    grp_oh = (gm == lax.broadcasted_iota(jnp.int32, (N_ROWS, N_GROUPS), 1)
              ).astype(jnp.float32)
    h_per = jnp.dot(per_oh, pt_ref[...], preferred_element_type=jnp.float32)
    h_grp = jnp.dot(grp_oh, gt_ref[...], preferred_element_type=jnp.float32)
    out_ref[...] = jnp.concatenate([zt_ref[...], h_per, h_grp], axis=-1)


def _fuse_tables(period_mapping, group_mapping, z_table, period_table,
                 group_table):
    return pl.pallas_call(
        _fuse_body,
        out_shape=jax.ShapeDtypeStruct((N_ROWS, D_OUT), jnp.float32),
    )(period_mapping.reshape(N_ROWS, 1), group_mapping.reshape(N_ROWS, 1),
      z_table, period_table, group_table)


@functools.lru_cache(maxsize=None)
def _make_gather(n_atoms):
    assert n_atoms % (_NW * _CHUNK) == 0
    rows_per_w = n_atoms // _NW
    n_chunks = rows_per_w // _CHUNK
    assert (n_chunks - 5) % 3 == 0
    n_super = (n_chunks - 5) // 3
    mesh = plsc.VectorSubcoreMesh(core_axis_name="c", subcore_axis_name="s")

    @functools.partial(
        pl.kernel,
        out_type=jax.ShapeDtypeStruct((n_atoms, D_OUT), jnp.float32),
        name="sc_embed_gather",
        mesh=mesh,
        compiler_params=pltpu.CompilerParams(needs_layout_passes=False),
        scratch_types=[
            pltpu.VMEM((rows_per_w + 16,), jnp.int32),
            pltpu.VMEM((N_ROWS, D_OUT), jnp.float32),
            pltpu.VMEM((_CHUNK, D_OUT), jnp.float32),
            pltpu.VMEM((_CHUNK, D_OUT), jnp.float32),
            pltpu.VMEM((_CHUNK, D_OUT), jnp.float32),
            pltpu.SemaphoreType.DMA,
            pltpu.SemaphoreType.DMA,
            pltpu.SemaphoreType.DMA,
            pltpu.SemaphoreType.DMA,
            pltpu.SemaphoreType.DMA,
            pltpu.SemaphoreType.DMA,
        ],
    )
    def gather(z_hbm, fused_rep_hbm, fused_hbm, out_hbm, idx_v, table_v,
               rows0, rows1, rows2, sg0, sg1, sg2, sw0, sw1, sw2):
        wid = lax.axis_index("s") * _NC + lax.axis_index("c")
        base = wid * rows_per_w
        rows = (rows0, rows1, rows2)
        sg = (sg0, sg1, sg2)
        sw = (sw0, sw1, sw2)

        def gather_desc(g, b):
            return pltpu.make_async_copy(
                fused_rep_hbm.at[idx_v.at[pl.ds(g * _CHUNK, _PRE)]],
                rows[b].at[pl.ds(0, _PRE)], sg[b])

        def write_desc(g, b):
            return pltpu.make_async_copy(
                rows[b], out_hbm.at[pl.ds(base + g * _CHUNK, _CHUNK)],
                sw[b])

        def fill(g, b):
            @plsc.parallel_loop(_PRE, _CHUNK, unroll=4)
            def fbody(a):
                zl = idx_v[pl.ds(g * _CHUNK + a, 16)]
                zi = (zl[0] >> 3) - wid * N_ROWS
                for kk in range(D_OUT // 16):
                    sl = pl.ds(kk * 16, 16)
                    rows[b][a, sl] = table_v[zi, sl]

        pltpu.sync_copy(fused_hbm, table_v)
        pltpu.sync_copy(z_hbm.at[pl.ds(base, rows_per_w)], idx_v.at[
            pl.ds(0, rows_per_w)])
        off = wid * (N_ROWS * _K)
        pat = lax.iota(jnp.int32, 16) & (_K - 1)

        def addoff(i, carry):
            sl = pl.ds(i * 16, 16)
            idx_v[sl] = idx_v[sl] * _K + (pat + off)
            return carry

        lax.fori_loop(0, rows_per_w // 16, addoff, 0)

        gather_desc(0, 0).start()
        fill(0, 0)
        gather_desc(1, 1).start()
        fill(1, 1)

        gather_desc(0, 0).wait()
        write_desc(0, 0).start()
        gather_desc(2, 2).start()
        fill(2, 2)

        def body(s, carry):
            for j in range(3):
                g = 1 + 3 * s + j
                b = (1 + j) % 3
                gather_desc(g, b).wait()
                write_desc(g, b).start()
                write_desc(g - 1, j).wait()
                gather_desc(g + 2, j).start()
                fill(g + 2, j)
            return carry

        lax.fori_loop(0, n_super, body, 0)

        for t in range(2):
            g = n_chunks - 4 + t
            b = g % 3
            gather_desc(g, b).wait()
            write_desc(g, b).start()
            write_desc(g - 1, (g - 1) % 3).wait()
            gather_desc(g + 2, (g - 1) % 3).start()
            fill(g + 2, (g - 1) % 3)

        for t in range(2):
            g = n_chunks - 2 + t
            b = g % 3
            gather_desc(g, b).wait()
            write_desc(g, b).start()

        for g in (n_chunks - 3, n_chunks - 2, n_chunks - 1):
            write_desc(g, g % 3).wait()

    return gather


def kernel(z, period_mapping, group_mapping, z_table, period_table,
           group_table):
    fused = _fuse_tables(period_mapping, group_mapping, z_table,
                         period_table, group_table)
    fused_rep = jnp.tile(jnp.repeat(fused, _K, axis=0), (_NW, 1))
    return _make_gather(N_ATOMS)(z, fused_rep, fused)
